# Initial kernel scaffold; baseline (speedup 1.0000x reference)
import jax
import jax.numpy as jnp
from jax.experimental import pallas as pl


def kernel(x, table):
    raise NotImplementedError("write your pallas kernel here")


# speedup vs baseline: 2.3624x; 2.3624x over previous
import functools
import math

import jax
import jax.numpy as jnp
from jax import lax
from jax.experimental import pallas as pl
from jax.experimental.pallas import tpu as pltpu
from jax.experimental.pallas import tpu_sc as plsc

D_MODEL = 128
SQRT_D = math.sqrt(D_MODEL)
CHUNK = 128


@functools.cache
def _make_gather(B: int, V: int, D: int):
    info = plsc.get_sparse_core_info()
    NC, NS = info.num_cores, info.num_subcores
    NW = NC * NS
    assert B % (NW * CHUNK) == 0
    b_per_w = B // NW
    n_chunks = b_per_w // CHUNK

    mesh = plsc.VectorSubcoreMesh(core_axis_name="c", subcore_axis_name="s")

    @functools.partial(
        pl.kernel,
        out_type=jax.ShapeDtypeStruct((B, D), jnp.float32),
        mesh=mesh,
        scratch_types=[
            pltpu.VMEM((CHUNK,), jnp.int32),
            pltpu.VMEM((CHUNK, D), jnp.float32),
            pltpu.SemaphoreType.DMA,
        ],
    )
    def gather_kernel(table_hbm, idx_hbm, out_hbm, idx_v, rows_v, sem):
        wid = lax.axis_index("s") * NC + lax.axis_index("c")
        wbase = wid * b_per_w

        def chunk_body(g, _):
            base = wbase + g * CHUNK
            pltpu.sync_copy(idx_hbm.at[pl.ds(base, CHUNK)], idx_v)
            pltpu.async_copy(table_hbm.at[idx_v], rows_v, sem).wait()

            def group_body(gr, _):
                iv = idx_v[pl.ds(gr * 16, 16)]
                scv = jnp.where(iv == 0, 0.0, SQRT_D)
                for r in range(16):
                    j = gr * 16 + r
                    sc = scv[r]
                    for k in range(D // 16):
                        sl = pl.ds(k * 16, 16)
                        rows_v[j, sl] = rows_v[j, sl] * sc
                return 0

            lax.fori_loop(0, CHUNK // 16, group_body, 0, unroll=False)
            pltpu.sync_copy(rows_v, out_hbm.at[pl.ds(base, CHUNK)])
            return 0

        lax.fori_loop(0, n_chunks, chunk_body, 0, unroll=False)

    return gather_kernel


def kernel(x, table):
    B = x.shape[0] * x.shape[1]
    V, D = table.shape
    idx = x.reshape(B).astype(jnp.int32)
    out = _make_gather(B, V, D)(table, idx)
    return out.reshape(x.shape[0], x.shape[1], D)


# speedup vs baseline: 3.0595x; 1.2951x over previous
import functools
import math

import jax
import jax.numpy as jnp
from jax import lax
from jax.experimental import pallas as pl
from jax.experimental.pallas import tpu as pltpu
from jax.experimental.pallas import tpu_sc as plsc

D_MODEL = 128
SQRT_D = math.sqrt(D_MODEL)
CHUNK = 128
NBUF = 5


@functools.cache
def _make_gather(B: int, V: int, D: int):
    info = plsc.get_sparse_core_info()
    NC, NS = info.num_cores, info.num_subcores
    NW = NC * NS
    assert B % (NW * CHUNK) == 0
    b_per_w = B // NW
    n_chunks = b_per_w // CHUNK
    assert n_chunks % NBUF == 0

    mesh = plsc.VectorSubcoreMesh(core_axis_name="c", subcore_axis_name="s")

    @functools.partial(
        pl.kernel,
        out_type=jax.ShapeDtypeStruct((B, D), jnp.float32),
        mesh=mesh,
        scratch_types=[
            pltpu.VMEM((b_per_w,), jnp.int32),
        ] + [pltpu.VMEM((CHUNK, D), jnp.float32) for _ in range(NBUF)] + [
            pltpu.SemaphoreType.DMA((NBUF,)),
            pltpu.SemaphoreType.DMA((NBUF,)),
        ],
    )
    def gather_kernel(table_hbm, idx_hbm, out_hbm, idx_all, *bufs_and_sems):
        rows = list(bufs_and_sems[:NBUF])
        gsem, osem = bufs_and_sems[NBUF], bufs_and_sems[NBUF + 1]
        wid = lax.axis_index("s") * NC + lax.axis_index("c")
        wbase = wid * b_per_w
        wchunk0 = wid * n_chunks

        pltpu.sync_copy(idx_hbm.at[pl.ds(wbase, b_per_w)], idx_all)

        def start_gather(g, b):
            pltpu.make_async_copy(
                table_hbm.at[idx_all.at[pl.ds(g * CHUNK, CHUNK)]],
                rows[b], gsem.at[b],
            ).start()

        def wait_gather(g, b):
            pltpu.make_async_copy(
                table_hbm.at[idx_all.at[pl.ds(g * CHUNK, CHUNK)]],
                rows[b], gsem.at[b],
            ).wait()

        def out_copy(g, b):
            return pltpu.make_async_copy(
                rows[b], out_hbm.at[pl.ds((wchunk0 + g) * CHUNK, CHUNK)],
                osem.at[b],
            )

        for b in range(NBUF - 1):
            start_gather(b, b)

        def outer(o, _):
            for b in range(NBUF):
                g = o * NBUF + b
                bb = (b + NBUF - 1) % NBUF
                wait_gather(g, b)

                def group_body(gr, _):
                    iv = idx_all[pl.ds(g * CHUNK + gr * 16, 16)]
                    scv = jnp.where(iv == 0, 0.0, SQRT_D)
                    for r in range(16):
                        j = gr * 16 + r
                        sc = scv[r]
                        for k in range(D // 16):
                            sl = pl.ds(k * 16, 16)
                            rows[b][j, sl] = rows[b][j, sl] * sc
                    return 0

                lax.fori_loop(0, CHUNK // 16, group_body, 0, unroll=False)

                nxt = g + NBUF - 1

                @pl.when(jnp.logical_and(g >= 1, nxt < n_chunks))
                def _():
                    out_copy(g - 1, bb).wait()

                @pl.when(nxt < n_chunks)
                def _():
                    start_gather(nxt, bb)

                out_copy(g, b).start()
            return 0

        lax.fori_loop(0, n_chunks // NBUF, outer, 0, unroll=False)

        for b in range(NBUF):
            out_copy(n_chunks - NBUF + b, b).wait()

    return gather_kernel


def kernel(x, table):
    B = x.shape[0] * x.shape[1]
    V, D = table.shape
    idx = x.reshape(B).astype(jnp.int32)
    out = _make_gather(B, V, D)(table, idx)
    return out.reshape(x.shape[0], x.shape[1], D)


# speedup vs baseline: 4.6144x; 1.5082x over previous
import functools
import math

import jax
import jax.numpy as jnp
from jax import lax
from jax.experimental import pallas as pl
from jax.experimental.pallas import tpu as pltpu
from jax.experimental.pallas import tpu_sc as plsc

D_MODEL = 128
SQRT_D = math.sqrt(D_MODEL)
NBUF = 4


@functools.cache
def _make_gather(NB: int, L: int, V: int, D: int):
    info = plsc.get_sparse_core_info()
    NC, NS = info.num_cores, info.num_subcores
    NW = NC * NS
    assert NB % NW == 0
    nb_w = NB // NW
    assert nb_w % NBUF == 0

    mesh = plsc.VectorSubcoreMesh(core_axis_name="c", subcore_axis_name="s")

    @functools.partial(
        pl.kernel,
        out_type=jax.ShapeDtypeStruct((NB, L, D), jnp.float32),
        mesh=mesh,
        scratch_types=[
            pltpu.VMEM((nb_w, L), jnp.int32),
        ] + [pltpu.VMEM((L, D), jnp.float32) for _ in range(NBUF)] + [
            pltpu.SemaphoreType.DMA((NBUF,)),
            pltpu.SemaphoreType.DMA((NBUF,)),
        ],
    )
    def gather_kernel(table_hbm, idx_hbm, out_hbm, idx_all, *bufs_and_sems):
        rows = list(bufs_and_sems[:NBUF])
        gsem, osem = bufs_and_sems[NBUF], bufs_and_sems[NBUF + 1]
        wid = lax.axis_index("s") * NC + lax.axis_index("c")
        wb0 = wid * nb_w

        pltpu.sync_copy(idx_hbm.at[pl.ds(wb0, nb_w)], idx_all)

        def start_gather(g, b):
            pltpu.make_async_copy(
                table_hbm.at[idx_all.at[g]], rows[b], gsem.at[b]
            ).start()

        def wait_gather(g, b):
            pltpu.make_async_copy(
                table_hbm.at[idx_all.at[g]], rows[b], gsem.at[b]
            ).wait()

        def out_copy(g, b):
            return pltpu.make_async_copy(
                rows[b], out_hbm.at[wb0 + g], osem.at[b]
            )

        for b in range(NBUF - 1):
            start_gather(b, b)

        def scale_rows(g, b, j0, iv_off, lanes):
            iv = idx_all[g, pl.ds(iv_off, 16)]
            scv = jnp.where(iv == 0, 0.0, SQRT_D)
            for i, r in enumerate(lanes):
                sc = scv[r]
                for k in range(D // 16):
                    sl = pl.ds(k * 16, 16)
                    rows[b][j0 + i, sl] = rows[b][j0 + i, sl] * sc

        def outer(o, _):
            for b in range(NBUF):
                g = o * NBUF + b
                bb = (b + NBUF - 1) % NBUF
                wait_gather(g, b)

                for j0 in range(0, (L // 16) * 16, 16):
                    scale_rows(g, b, j0, j0, range(16))
                if L % 16:
                    scale_rows(g, b, (L // 16) * 16, L - 16,
                               range(16 - L % 16, 16))

                nxt = g + NBUF - 1

                @pl.when(jnp.logical_and(g >= 1, nxt < nb_w))
                def _():
                    out_copy(g - 1, bb).wait()

                @pl.when(nxt < nb_w)
                def _():
                    start_gather(nxt, bb)

                out_copy(g, b).start()
            return 0

        lax.fori_loop(0, nb_w // NBUF, outer, 0, unroll=False)

        for b in range(NBUF):
            out_copy(nb_w - NBUF + b, b).wait()

    return gather_kernel


def kernel(x, table):
    NB, L = x.shape
    V, D = table.shape
    return _make_gather(NB, L, V, D)(table, x.astype(jnp.int32))


# speedup vs baseline: 9.5886x; 2.0780x over previous
import functools
import math

import jax
import jax.numpy as jnp
from jax import lax
from jax.experimental import pallas as pl
from jax.experimental.pallas import tpu as pltpu
from jax.experimental.pallas import tpu_sc as plsc

D_MODEL = 128
SQRT_D = math.sqrt(D_MODEL)
NBUF = 5
CB = 128


@functools.cache
def _make_gather(NB: int, L: int, V: int, D: int):
    info = plsc.get_sparse_core_info()
    NC, NS = info.num_cores, info.num_subcores
    NW = NC * NS
    assert NB == NW * CB and L % NBUF == 0

    mesh = plsc.VectorSubcoreMesh(core_axis_name="c", subcore_axis_name="s")

    @functools.partial(
        pl.kernel,
        out_type=jax.ShapeDtypeStruct((L, NB, D), jnp.float32),
        mesh=mesh,
        scratch_types=[
            pltpu.VMEM((L, CB), jnp.int32),
        ] + [pltpu.VMEM((CB, D), jnp.float32) for _ in range(NBUF)] + [
            pltpu.SemaphoreType.DMA((NBUF,)),
            pltpu.SemaphoreType.DMA((NBUF,)),
        ],
    )
    def gather_kernel(table_hbm, idx_hbm, out_hbm, idx_all, *bufs_and_sems):
        rows = list(bufs_and_sems[:NBUF])
        gsem, osem = bufs_and_sems[NBUF], bufs_and_sems[NBUF + 1]
        wid = lax.axis_index("s") * NC + lax.axis_index("c")

        pltpu.sync_copy(idx_hbm.at[:, wid], idx_all)

        def start_gather(g, b):
            pltpu.make_async_copy(
                table_hbm.at[idx_all.at[g]], rows[b], gsem.at[b]
            ).start()

        def wait_gather(g, b):
            pltpu.make_async_copy(
                table_hbm.at[idx_all.at[g]], rows[b], gsem.at[b]
            ).wait()

        def out_copy(g, b):
            return pltpu.make_async_copy(
                rows[b], out_hbm.at[g, pl.ds(wid * CB, CB)], osem.at[b]
            )

        for b in range(NBUF - 1):
            start_gather(b, b)

        def outer(o, _):
            for b in range(NBUF):
                g = o * NBUF + b
                bb = (b + NBUF - 1) % NBUF
                wait_gather(g, b)

                def group_body(gr, _):
                    iv = idx_all[g, pl.ds(gr * 16, 16)]
                    scv = jnp.where(iv == 0, 0.0, SQRT_D)
                    for r in range(16):
                        j = gr * 16 + r
                        sc = scv[r]
                        for k in range(D // 16):
                            sl = pl.ds(k * 16, 16)
                            rows[b][j, sl] = rows[b][j, sl] * sc
                    return 0

                lax.fori_loop(0, CB // 16, group_body, 0, unroll=False)

                nxt = g + NBUF - 1

                @pl.when(jnp.logical_and(g >= 1, nxt < L))
                def _():
                    out_copy(g - 1, bb).wait()

                @pl.when(nxt < L)
                def _():
                    start_gather(nxt, bb)

                out_copy(g, b).start()
            return 0

        lax.fori_loop(0, L // NBUF, outer, 0, unroll=False)

        for b in range(NBUF):
            out_copy(L - NBUF + b, b).wait()

    return gather_kernel


def kernel(x, table):
    NB, L = x.shape
    V, D = table.shape
    xt = x.T.reshape(L, NB // CB, CB).astype(jnp.int32)
    out = _make_gather(NB, L, V, D)(table, xt)
    return out.transpose(1, 0, 2)
